# baseline (speedup 1.0000x reference)
import functools

import jax
import jax.numpy as jnp
from jax import lax
from jax.experimental import pallas as pl
from jax.experimental.pallas import tpu as pltpu
from jax.experimental.pallas import tpu_sc as plsc

NC = 2
NS = 16
NW = NC * NS
LANES = 16
K = 80

_F32 = jnp.float32
_BF16 = jnp.bfloat16


def _sc_mesh():
    return plsc.VectorSubcoreMesh(
        core_axis_name="c", subcore_axis_name="s",
        num_cores=NC, num_subcores=NS)


NB = 5


def _pick_chunk(ew, kmax, step=8):
    for k in range(kmax - kmax % step, 0, -step):
        if ew % k == 0 and (ew // k) % NB == 0:
            return k
    raise ValueError(ew)


@functools.lru_cache(maxsize=None)
def _make_gather_add(n, e, h):
    ew = e // NW
    k = _pick_chunk(ew, 504 // NB)
    nchunk = ew // k
    ngroups = nchunk // NB

    @functools.partial(
        pl.kernel,
        mesh=_sc_mesh(),
        out_type=jax.ShapeDtypeStruct((e, h), _F32),
        scratch_types=[
            pltpu.VMEM((NB, k), jnp.int32),
            pltpu.VMEM((NB, k), jnp.int32),
            pltpu.VMEM((NB, k, h), _F32),
            pltpu.VMEM((NB, k, h), _F32),
            pltpu.SemaphoreType.DMA((NB,)),
            pltpu.SemaphoreType.DMA((NB,)),
            pltpu.SemaphoreType.DMA((NB,)),
            pltpu.SemaphoreType.DMA((NB,)),
            pltpu.SemaphoreType.DMA((NB,)),
        ],
    )
    def gather_add(a_hbm, b_hbm, src_hbm, dst_hbm, out_hbm,
                   sidx, didx, arows, brows,
                   sem_s, sem_d, sem_a, sem_b, sem_o):
        wid = lax.axis_index("s") * NC + lax.axis_index("c")
        base = wid * ew

        def group(g, carry):
            cs, cd = [], []
            for b in range(NB):
                off = base + (g * NB + b) * k
                cs.append(pltpu.async_copy(
                    src_hbm.at[pl.ds(off, k)], sidx.at[b], sem_s.at[b]))
                cd.append(pltpu.async_copy(
                    dst_hbm.at[pl.ds(off, k)], didx.at[b], sem_d.at[b]))
            ga, gb = [], []
            for b in range(NB):
                cs[b].wait()
                cd[b].wait()

                @pl.when(g > 0)
                def _(b=b):
                    off_prev = base + ((g - 1) * NB + b) * k
                    pltpu.make_async_copy(
                        arows.at[b], out_hbm.at[pl.ds(off_prev, k)],
                        sem_o.at[b]).wait()

                ga.append(pltpu.async_copy(
                    a_hbm.at[sidx.at[b]], arows.at[b], sem_a.at[b]))
                gb.append(pltpu.async_copy(
                    b_hbm.at[didx.at[b]], brows.at[b], sem_b.at[b]))
            for b in range(NB):
                ga[b].wait()
                gb[b].wait()

                def row(r, c2, b=b):
                    for j in range(h // LANES):
                        sl = pl.ds(j * LANES, LANES)
                        plsc.addupdate(arows.at[b, r, sl], brows[b, r, sl])
                    return c2

                lax.fori_loop(0, k, row, 0, unroll=4)
                off = base + (g * NB + b) * k
                pltpu.async_copy(
                    arows.at[b], out_hbm.at[pl.ds(off, k)], sem_o.at[b])
            return carry

        lax.fori_loop(0, ngroups, group, 0)
        for b in range(NB):
            off_last = base + ((ngroups - 1) * NB + b) * k
            pltpu.make_async_copy(
                arows.at[b], out_hbm.at[pl.ds(off_last, k)],
                sem_o.at[b]).wait()

    return gather_add


@functools.lru_cache(maxsize=None)
def _make_scatter_add(n, e, h):
    ew = e // NW
    ks = _pick_chunk(ew, 72)
    nchunk = ew // ks
    zr = min(ks, 40)
    rpt = -(-(-(-n // NS)) // zr) * zr
    last_rows = n - (NS - 1) * rpt
    assert rpt % zr == 0 and last_rows % zr == 0 and last_rows > 0
    assert rpt % 8 == 0 and last_rows % 8 == 0

    @functools.partial(
        pl.kernel,
        mesh=_sc_mesh(),
        out_type=jax.ShapeDtypeStruct((NC, n, h), _F32),
        scratch_types=[
            pltpu.VMEM((NB, ks), jnp.int32),
            pltpu.VMEM((NB, ks, h), _F32),
            pltpu.VMEM_SHARED((n, h), _F32),
            pltpu.SemaphoreType.DMA((NB,)),
            pltpu.SemaphoreType.DMA((NB,)),
        ],
    )
    def scatter_add(m_hbm, dst_hbm, out_hbm, didx, rows, acc,
                    sem_i, sem_r):
        cid = lax.axis_index("c")
        sid = lax.axis_index("s")
        wid = sid * NC + cid

        zero = jnp.zeros((LANES,), _F32)
        zbuf = rows.at[0, pl.ds(0, zr)]

        def zrow(r, carry):
            for j in range(h // LANES):
                rows[0, r, pl.ds(j * LANES, LANES)] = zero
            return carry

        lax.fori_loop(0, zr, zrow, 0)
        tb = sid * rpt
        nrows = jnp.where(sid == NS - 1, last_rows, rpt)
        ncopies = nrows // zr

        def zcopy(i, carry):
            pltpu.sync_copy(zbuf, acc.at[pl.ds(tb + i * zr, zr)])
            return carry

        lax.fori_loop(0, ncopies, zcopy, 0)
        plsc.subcore_barrier()

        base = wid * ew
        ngroups = nchunk // NB

        def group(g, carry):
            ci_, cr_ = [], []
            for b in range(NB):
                off = base + (g * NB + b) * ks
                ci_.append(pltpu.async_copy(
                    dst_hbm.at[pl.ds(off, ks)], didx.at[b], sem_i.at[b]))
                cr_.append(pltpu.async_copy(
                    m_hbm.at[pl.ds(off, ks)], rows.at[b], sem_r.at[b]))
            for b in range(NB):
                ci_[b].wait()
                cr_[b].wait()
                pltpu.sync_copy(rows.at[b], acc.at[didx.at[b]], add=True)
            return carry

        lax.fori_loop(0, ngroups, group, 0)
        plsc.subcore_barrier()

        def ocopy(i, carry):
            sl = pl.ds(tb + i * zr, zr)
            pltpu.sync_copy(acc.at[sl], out_hbm.at[cid].at[sl])
            return carry

        lax.fori_loop(0, ncopies, ocopy, 0)

    return scatter_add


def _proj2_body(x_ref, ws_ref, wd_ref, a_ref, b_ref):
    x = x_ref[...]
    a_ref[...] = jnp.dot(x, ws_ref[...], preferred_element_type=_F32)
    b_ref[...] = jnp.dot(x, wd_ref[...], preferred_element_type=_F32)


def _proj2(x, ws, wd, bn=2000):
    n, d = x.shape
    h = ws.shape[1]
    grid = n // bn
    return pl.pallas_call(
        _proj2_body,
        grid=(grid,),
        in_specs=[
            pl.BlockSpec((bn, d), lambda i: (i, 0)),
            pl.BlockSpec((d, h), lambda i: (0, 0)),
            pl.BlockSpec((d, h), lambda i: (0, 0)),
        ],
        out_specs=[
            pl.BlockSpec((bn, h), lambda i: (i, 0)),
            pl.BlockSpec((bn, h), lambda i: (i, 0)),
        ],
        out_shape=[
            jax.ShapeDtypeStruct((n, h), _F32),
            jax.ShapeDtypeStruct((n, h), _F32),
        ],
    )(x, ws, wd)


def _edge_mlp_body(g_ref, ea_ref, w1e_ref, b1_ref, w2_ref, b2_ref, out_ref):
    c = jnp.dot(ea_ref[...], w1e_ref[...], preferred_element_type=_F32)
    m1 = jnp.maximum(g_ref[...] + c + b1_ref[...], 0.0)
    m2 = jnp.dot(m1, w2_ref[...], preferred_element_type=_F32)
    out_ref[...] = jnp.maximum(m2 + b2_ref[...], 0.0)


def _edge_mlp(g, ea, w1e, b1, w2, b2, be=3200):
    e, h = g.shape
    de = ea.shape[1]
    grid = e // be
    return pl.pallas_call(
        _edge_mlp_body,
        grid=(grid,),
        in_specs=[
            pl.BlockSpec((be, h), lambda i: (i, 0)),
            pl.BlockSpec((be, de), lambda i: (i, 0)),
            pl.BlockSpec((de, h), lambda i: (0, 0)),
            pl.BlockSpec((1, h), lambda i: (0, 0)),
            pl.BlockSpec((h, h), lambda i: (0, 0)),
            pl.BlockSpec((1, h), lambda i: (0, 0)),
        ],
        out_specs=pl.BlockSpec((be, h), lambda i: (i, 0)),
        out_shape=jax.ShapeDtypeStruct((e, h), _F32),
    )(g, ea, w1e, b1.reshape(1, h), w2, b2.reshape(1, h))


def _node_mid_body(x_ref, p_ref, q_ref, w1a_ref, w1b_ref, nb1_ref, w2_ref,
                   nb2_ref, ws2_ref, wd2_ref, h_ref, a2_ref, b2_ref):
    agg = (p_ref[0] + p_ref[1]) + (q_ref[0] + q_ref[1])
    t = jnp.dot(x_ref[...], w1a_ref[...], preferred_element_type=_F32)
    t = t + jnp.dot(agg, w1b_ref[...], preferred_element_type=_F32)
    t = jnp.maximum(t + nb1_ref[...], 0.0)
    hh = jnp.dot(t, w2_ref[...], preferred_element_type=_F32) + nb2_ref[...]
    h_ref[...] = hh
    a2_ref[...] = jnp.dot(hh, ws2_ref[...], preferred_element_type=_F32)
    b2_ref[...] = jnp.dot(hh, wd2_ref[...], preferred_element_type=_F32)


def _node_mid(x, p, q, w1a, w1b, nb1, w2, nb2, ws2, wd2, bn=2000):
    n, d = x.shape
    h = w1b.shape[0]
    grid = n // bn
    return pl.pallas_call(
        _node_mid_body,
        grid=(grid,),
        in_specs=[
            pl.BlockSpec((bn, d), lambda i: (i, 0)),
            pl.BlockSpec((NC, bn, h), lambda i: (0, i, 0)),
            pl.BlockSpec((NC, bn, h), lambda i: (0, i, 0)),
            pl.BlockSpec((d, h), lambda i: (0, 0)),
            pl.BlockSpec((h, h), lambda i: (0, 0)),
            pl.BlockSpec((1, h), lambda i: (0, 0)),
            pl.BlockSpec((h, h), lambda i: (0, 0)),
            pl.BlockSpec((1, h), lambda i: (0, 0)),
            pl.BlockSpec((h, h), lambda i: (0, 0)),
            pl.BlockSpec((h, h), lambda i: (0, 0)),
        ],
        out_specs=[
            pl.BlockSpec((bn, h), lambda i: (i, 0)),
            pl.BlockSpec((bn, h), lambda i: (i, 0)),
            pl.BlockSpec((bn, h), lambda i: (i, 0)),
        ],
        out_shape=[
            jax.ShapeDtypeStruct((n, h), _F32),
            jax.ShapeDtypeStruct((n, h), _F32),
            jax.ShapeDtypeStruct((n, h), _F32),
        ],
    )(x, p, q, w1a, w1b, nb1.reshape(1, h), w2, nb2.reshape(1, h), ws2, wd2)


def _node_final_body(x_ref, p_ref, q_ref, w1a_ref, w1b_ref, nb1_ref, w2_ref,
                     nb2_ref, hw1_ref, hb1_ref, hw2_ref, hb2_ref, out_ref):
    agg = (p_ref[0] + p_ref[1]) + (q_ref[0] + q_ref[1])
    t = jnp.dot(x_ref[...], w1a_ref[...], preferred_element_type=_F32)
    t = t + jnp.dot(agg, w1b_ref[...], preferred_element_type=_F32)
    t = jnp.maximum(t + nb1_ref[...], 0.0)
    hh = jnp.dot(t, w2_ref[...], preferred_element_type=_F32) + nb2_ref[...]
    z = jnp.maximum(
        jnp.dot(hh, hw1_ref[...], preferred_element_type=_F32) + hb1_ref[...],
        0.0)
    out_ref[...] = (
        jnp.dot(z, hw2_ref[...], preferred_element_type=_F32) + hb2_ref[...])


def _node_final(x, p, q, w1a, w1b, nb1, w2, nb2, hw1, hb1, hw2, hb2, bn=2000):
    n, d = x.shape
    h = w1b.shape[0]
    grid = n // bn
    return pl.pallas_call(
        _node_final_body,
        grid=(grid,),
        in_specs=[
            pl.BlockSpec((bn, d), lambda i: (i, 0)),
            pl.BlockSpec((NC, bn, h), lambda i: (0, i, 0)),
            pl.BlockSpec((NC, bn, h), lambda i: (0, i, 0)),
            pl.BlockSpec((d, h), lambda i: (0, 0)),
            pl.BlockSpec((h, h), lambda i: (0, 0)),
            pl.BlockSpec((1, h), lambda i: (0, 0)),
            pl.BlockSpec((h, h), lambda i: (0, 0)),
            pl.BlockSpec((1, h), lambda i: (0, 0)),
            pl.BlockSpec((h, h), lambda i: (0, 0)),
            pl.BlockSpec((1, h), lambda i: (0, 0)),
            pl.BlockSpec((h, 1), lambda i: (0, 0)),
            pl.BlockSpec((1, 1), lambda i: (0, 0)),
        ],
        out_specs=pl.BlockSpec((bn, 1), lambda i: (i, 0)),
        out_shape=jax.ShapeDtypeStruct((n, 1), _F32),
    )(x, p, q, w1a, w1b, nb1.reshape(1, h), w2, nb2.reshape(1, h),
      hw1, hb1.reshape(1, h), hw2, hb2.reshape(1, 1))


def kernel(x, edge_index, edge_attr, params):
    n, d = x.shape
    e = edge_index.shape[1]
    ew_tot = e // NW
    ew_a = ((ew_tot * 16) // 25 // 400) * 400
    ea_n = NW * ew_a
    src_a, src_b = edge_index[0, :ea_n], edge_index[0, ea_n:]
    dst_a, dst_b = edge_index[1, :ea_n], edge_index[1, ea_n:]
    ea_a, ea_b = edge_attr[:ea_n], edge_attr[ea_n:]

    conv1, conv2 = params["convs"]
    head = params["head"]
    h = head["W1"].shape[0]

    gather_a = _make_gather_add(n, ea_n, h)
    gather_b = _make_gather_add(n, e - ea_n, h)
    scatter_a = _make_scatter_add(n, ea_n, h)
    scatter_b = _make_scatter_add(n, e - ea_n, h)

    def layer(xin, conv):
        ewt = conv["edge"]["W1"]
        dd = xin.shape[1]
        at, bt = _proj2(xin, ewt[:dd], ewt[dd:2 * dd])
        gb = gather_b(at, bt, src_b, dst_b)
        ga = gather_a(at, bt, src_a, dst_a)
        mb = _edge_mlp(gb, ea_b, ewt[2 * dd:], conv["edge"]["b1"],
                       conv["edge"]["W2"], conv["edge"]["b2"])
        ma = _edge_mlp(ga, ea_a, ewt[2 * dd:], conv["edge"]["b1"],
                       conv["edge"]["W2"], conv["edge"]["b2"])
        pb = scatter_b(mb, dst_b)
        pa = scatter_a(ma, dst_a)
        return pa, pb

    p1a, p1b = layer(x, conv1)
    nw1 = conv1["node"]["W1"]
    ew2 = conv2["edge"]["W1"]
    h1, a2t, b2t = _node_mid(
        x, p1a, p1b, nw1[:d], nw1[d:], conv1["node"]["b1"],
        conv1["node"]["W2"], conv1["node"]["b2"],
        ew2[:h], ew2[h:2 * h])

    g2b = gather_b(a2t, b2t, src_b, dst_b)
    g2a = gather_a(a2t, b2t, src_a, dst_a)
    m2b = _edge_mlp(g2b, ea_b, ew2[2 * h:], conv2["edge"]["b1"],
                    conv2["edge"]["W2"], conv2["edge"]["b2"])
    m2a = _edge_mlp(g2a, ea_a, ew2[2 * h:], conv2["edge"]["b1"],
                    conv2["edge"]["W2"], conv2["edge"]["b2"])
    p2b = scatter_b(m2b, dst_b)
    p2a = scatter_a(m2a, dst_a)

    nw2 = conv2["node"]["W1"]
    out = _node_final(
        h1, p2a, p2b, nw2[:h], nw2[h:], conv2["node"]["b1"],
        conv2["node"]["W2"], conv2["node"]["b2"],
        head["W1"], head["b1"], head["W2"], head["b2"])
    return out


# speedup vs baseline: 1.0005x; 1.0005x over previous
import functools

import jax
import jax.numpy as jnp
from jax import lax
from jax.experimental import pallas as pl
from jax.experimental.pallas import tpu as pltpu
from jax.experimental.pallas import tpu_sc as plsc

NC = 2
NS = 16
NW = NC * NS
LANES = 16
K = 80

_F32 = jnp.float32
_BF16 = jnp.bfloat16


def _sc_mesh():
    return plsc.VectorSubcoreMesh(
        core_axis_name="c", subcore_axis_name="s",
        num_cores=NC, num_subcores=NS)


NB = 5


def _pick_chunk(ew, kmax, step=8):
    for k in range(kmax - kmax % step, 0, -step):
        if ew % k == 0 and (ew // k) % NB == 0:
            return k
    raise ValueError(ew)


@functools.lru_cache(maxsize=None)
def _make_gather_add(n, e, h):
    ew = e // NW
    k = _pick_chunk(ew, 504 // NB)
    nchunk = ew // k
    ngroups = nchunk // NB

    @functools.partial(
        pl.kernel,
        mesh=_sc_mesh(),
        out_type=jax.ShapeDtypeStruct((e, h), _F32),
        scratch_types=[
            pltpu.VMEM((NB, k), jnp.int32),
            pltpu.VMEM((NB, k), jnp.int32),
            pltpu.VMEM((NB, k, h), _F32),
            pltpu.VMEM((NB, k, h), _F32),
            pltpu.SemaphoreType.DMA((NB,)),
            pltpu.SemaphoreType.DMA((NB,)),
            pltpu.SemaphoreType.DMA((NB,)),
            pltpu.SemaphoreType.DMA((NB,)),
            pltpu.SemaphoreType.DMA((NB,)),
        ],
    )
    def gather_add(a_hbm, b_hbm, src_hbm, dst_hbm, out_hbm,
                   sidx, didx, arows, brows,
                   sem_s, sem_d, sem_a, sem_b, sem_o):
        wid = lax.axis_index("s") * NC + lax.axis_index("c")
        base = wid * ew

        def group(g, carry):
            cs, cd = [], []
            for b in range(NB):
                off = base + (g * NB + b) * k
                cs.append(pltpu.async_copy(
                    src_hbm.at[pl.ds(off, k)], sidx.at[b], sem_s.at[b]))
                cd.append(pltpu.async_copy(
                    dst_hbm.at[pl.ds(off, k)], didx.at[b], sem_d.at[b]))
            ga, gb = [], []
            for b in range(NB):
                cs[b].wait()
                cd[b].wait()

                @pl.when(g > 0)
                def _(b=b):
                    off_prev = base + ((g - 1) * NB + b) * k
                    pltpu.make_async_copy(
                        arows.at[b], out_hbm.at[pl.ds(off_prev, k)],
                        sem_o.at[b]).wait()

                ga.append(pltpu.async_copy(
                    a_hbm.at[sidx.at[b]], arows.at[b], sem_a.at[b]))
                gb.append(pltpu.async_copy(
                    b_hbm.at[didx.at[b]], brows.at[b], sem_b.at[b]))
            for b in range(NB):
                ga[b].wait()
                gb[b].wait()

                def row(r, c2, b=b):
                    for j in range(h // LANES):
                        sl = pl.ds(j * LANES, LANES)
                        plsc.addupdate(arows.at[b, r, sl], brows[b, r, sl])
                    return c2

                lax.fori_loop(0, k, row, 0, unroll=2)
                off = base + (g * NB + b) * k
                pltpu.async_copy(
                    arows.at[b], out_hbm.at[pl.ds(off, k)], sem_o.at[b])
            return carry

        lax.fori_loop(0, ngroups, group, 0)
        for b in range(NB):
            off_last = base + ((ngroups - 1) * NB + b) * k
            pltpu.make_async_copy(
                arows.at[b], out_hbm.at[pl.ds(off_last, k)],
                sem_o.at[b]).wait()

    return gather_add


@functools.lru_cache(maxsize=None)
def _make_scatter_add(n, e, h):
    ew = e // NW
    ks = _pick_chunk(ew, 72)
    nchunk = ew // ks
    zr = min(ks, 40)
    rpt = -(-(-(-n // NS)) // zr) * zr
    last_rows = n - (NS - 1) * rpt
    assert rpt % zr == 0 and last_rows % zr == 0 and last_rows > 0
    assert rpt % 8 == 0 and last_rows % 8 == 0

    @functools.partial(
        pl.kernel,
        mesh=_sc_mesh(),
        out_type=jax.ShapeDtypeStruct((NC, n, h), _F32),
        scratch_types=[
            pltpu.VMEM((NB, ks), jnp.int32),
            pltpu.VMEM((NB, ks, h), _F32),
            pltpu.VMEM_SHARED((n, h), _F32),
            pltpu.SemaphoreType.DMA((NB,)),
            pltpu.SemaphoreType.DMA((NB,)),
        ],
    )
    def scatter_add(m_hbm, dst_hbm, out_hbm, didx, rows, acc,
                    sem_i, sem_r):
        cid = lax.axis_index("c")
        sid = lax.axis_index("s")
        wid = sid * NC + cid

        zero = jnp.zeros((LANES,), _F32)
        zbuf = rows.at[0, pl.ds(0, zr)]

        def zrow(r, carry):
            for j in range(h // LANES):
                rows[0, r, pl.ds(j * LANES, LANES)] = zero
            return carry

        lax.fori_loop(0, zr, zrow, 0)
        tb = sid * rpt
        nrows = jnp.where(sid == NS - 1, last_rows, rpt)
        ncopies = nrows // zr

        def zcopy(i, carry):
            pltpu.sync_copy(zbuf, acc.at[pl.ds(tb + i * zr, zr)])
            return carry

        lax.fori_loop(0, ncopies, zcopy, 0)
        plsc.subcore_barrier()

        base = wid * ew
        ngroups = nchunk // NB

        def group(g, carry):
            ci_, cr_ = [], []
            for b in range(NB):
                off = base + (g * NB + b) * ks
                ci_.append(pltpu.async_copy(
                    dst_hbm.at[pl.ds(off, ks)], didx.at[b], sem_i.at[b]))
                cr_.append(pltpu.async_copy(
                    m_hbm.at[pl.ds(off, ks)], rows.at[b], sem_r.at[b]))
            for b in range(NB):
                ci_[b].wait()
                cr_[b].wait()
                pltpu.sync_copy(rows.at[b], acc.at[didx.at[b]], add=True)
            return carry

        lax.fori_loop(0, ngroups, group, 0)
        plsc.subcore_barrier()

        def ocopy(i, carry):
            sl = pl.ds(tb + i * zr, zr)
            pltpu.sync_copy(acc.at[sl], out_hbm.at[cid].at[sl])
            return carry

        lax.fori_loop(0, ncopies, ocopy, 0)

    return scatter_add


def _proj2_body(x_ref, ws_ref, wd_ref, a_ref, b_ref):
    x = x_ref[...]
    a_ref[...] = jnp.dot(x, ws_ref[...], preferred_element_type=_F32)
    b_ref[...] = jnp.dot(x, wd_ref[...], preferred_element_type=_F32)


def _proj2(x, ws, wd, bn=2000):
    n, d = x.shape
    h = ws.shape[1]
    grid = n // bn
    return pl.pallas_call(
        _proj2_body,
        grid=(grid,),
        in_specs=[
            pl.BlockSpec((bn, d), lambda i: (i, 0)),
            pl.BlockSpec((d, h), lambda i: (0, 0)),
            pl.BlockSpec((d, h), lambda i: (0, 0)),
        ],
        out_specs=[
            pl.BlockSpec((bn, h), lambda i: (i, 0)),
            pl.BlockSpec((bn, h), lambda i: (i, 0)),
        ],
        out_shape=[
            jax.ShapeDtypeStruct((n, h), _F32),
            jax.ShapeDtypeStruct((n, h), _F32),
        ],
    )(x, ws, wd)


def _edge_mlp_body(g_ref, ea_ref, w1e_ref, b1_ref, w2_ref, b2_ref, out_ref):
    c = jnp.dot(ea_ref[...], w1e_ref[...], preferred_element_type=_F32)
    m1 = jnp.maximum(g_ref[...] + c + b1_ref[...], 0.0)
    m2 = jnp.dot(m1, w2_ref[...], preferred_element_type=_F32)
    out_ref[...] = jnp.maximum(m2 + b2_ref[...], 0.0)


def _edge_mlp(g, ea, w1e, b1, w2, b2, be=3200):
    e, h = g.shape
    de = ea.shape[1]
    grid = e // be
    return pl.pallas_call(
        _edge_mlp_body,
        grid=(grid,),
        in_specs=[
            pl.BlockSpec((be, h), lambda i: (i, 0)),
            pl.BlockSpec((be, de), lambda i: (i, 0)),
            pl.BlockSpec((de, h), lambda i: (0, 0)),
            pl.BlockSpec((1, h), lambda i: (0, 0)),
            pl.BlockSpec((h, h), lambda i: (0, 0)),
            pl.BlockSpec((1, h), lambda i: (0, 0)),
        ],
        out_specs=pl.BlockSpec((be, h), lambda i: (i, 0)),
        out_shape=jax.ShapeDtypeStruct((e, h), _F32),
    )(g, ea, w1e, b1.reshape(1, h), w2, b2.reshape(1, h))


def _node_mid_body(x_ref, p_ref, q_ref, w1a_ref, w1b_ref, nb1_ref, w2_ref,
                   nb2_ref, ws2_ref, wd2_ref, h_ref, a2_ref, b2_ref):
    agg = (p_ref[0] + p_ref[1]) + (q_ref[0] + q_ref[1])
    t = jnp.dot(x_ref[...], w1a_ref[...], preferred_element_type=_F32)
    t = t + jnp.dot(agg, w1b_ref[...], preferred_element_type=_F32)
    t = jnp.maximum(t + nb1_ref[...], 0.0)
    hh = jnp.dot(t, w2_ref[...], preferred_element_type=_F32) + nb2_ref[...]
    h_ref[...] = hh
    a2_ref[...] = jnp.dot(hh, ws2_ref[...], preferred_element_type=_F32)
    b2_ref[...] = jnp.dot(hh, wd2_ref[...], preferred_element_type=_F32)


def _node_mid(x, p, q, w1a, w1b, nb1, w2, nb2, ws2, wd2, bn=2000):
    n, d = x.shape
    h = w1b.shape[0]
    grid = n // bn
    return pl.pallas_call(
        _node_mid_body,
        grid=(grid,),
        in_specs=[
            pl.BlockSpec((bn, d), lambda i: (i, 0)),
            pl.BlockSpec((NC, bn, h), lambda i: (0, i, 0)),
            pl.BlockSpec((NC, bn, h), lambda i: (0, i, 0)),
            pl.BlockSpec((d, h), lambda i: (0, 0)),
            pl.BlockSpec((h, h), lambda i: (0, 0)),
            pl.BlockSpec((1, h), lambda i: (0, 0)),
            pl.BlockSpec((h, h), lambda i: (0, 0)),
            pl.BlockSpec((1, h), lambda i: (0, 0)),
            pl.BlockSpec((h, h), lambda i: (0, 0)),
            pl.BlockSpec((h, h), lambda i: (0, 0)),
        ],
        out_specs=[
            pl.BlockSpec((bn, h), lambda i: (i, 0)),
            pl.BlockSpec((bn, h), lambda i: (i, 0)),
            pl.BlockSpec((bn, h), lambda i: (i, 0)),
        ],
        out_shape=[
            jax.ShapeDtypeStruct((n, h), _F32),
            jax.ShapeDtypeStruct((n, h), _F32),
            jax.ShapeDtypeStruct((n, h), _F32),
        ],
    )(x, p, q, w1a, w1b, nb1.reshape(1, h), w2, nb2.reshape(1, h), ws2, wd2)


def _node_final_body(x_ref, p_ref, q_ref, w1a_ref, w1b_ref, nb1_ref, w2_ref,
                     nb2_ref, hw1_ref, hb1_ref, hw2_ref, hb2_ref, out_ref):
    agg = (p_ref[0] + p_ref[1]) + (q_ref[0] + q_ref[1])
    t = jnp.dot(x_ref[...], w1a_ref[...], preferred_element_type=_F32)
    t = t + jnp.dot(agg, w1b_ref[...], preferred_element_type=_F32)
    t = jnp.maximum(t + nb1_ref[...], 0.0)
    hh = jnp.dot(t, w2_ref[...], preferred_element_type=_F32) + nb2_ref[...]
    z = jnp.maximum(
        jnp.dot(hh, hw1_ref[...], preferred_element_type=_F32) + hb1_ref[...],
        0.0)
    out_ref[...] = (
        jnp.dot(z, hw2_ref[...], preferred_element_type=_F32) + hb2_ref[...])


def _node_final(x, p, q, w1a, w1b, nb1, w2, nb2, hw1, hb1, hw2, hb2, bn=2000):
    n, d = x.shape
    h = w1b.shape[0]
    grid = n // bn
    return pl.pallas_call(
        _node_final_body,
        grid=(grid,),
        in_specs=[
            pl.BlockSpec((bn, d), lambda i: (i, 0)),
            pl.BlockSpec((NC, bn, h), lambda i: (0, i, 0)),
            pl.BlockSpec((NC, bn, h), lambda i: (0, i, 0)),
            pl.BlockSpec((d, h), lambda i: (0, 0)),
            pl.BlockSpec((h, h), lambda i: (0, 0)),
            pl.BlockSpec((1, h), lambda i: (0, 0)),
            pl.BlockSpec((h, h), lambda i: (0, 0)),
            pl.BlockSpec((1, h), lambda i: (0, 0)),
            pl.BlockSpec((h, h), lambda i: (0, 0)),
            pl.BlockSpec((1, h), lambda i: (0, 0)),
            pl.BlockSpec((h, 1), lambda i: (0, 0)),
            pl.BlockSpec((1, 1), lambda i: (0, 0)),
        ],
        out_specs=pl.BlockSpec((bn, 1), lambda i: (i, 0)),
        out_shape=jax.ShapeDtypeStruct((n, 1), _F32),
    )(x, p, q, w1a, w1b, nb1.reshape(1, h), w2, nb2.reshape(1, h),
      hw1, hb1.reshape(1, h), hw2, hb2.reshape(1, 1))


def kernel(x, edge_index, edge_attr, params):
    n, d = x.shape
    e = edge_index.shape[1]
    ew_tot = e // NW
    ew_a = ((ew_tot * 16) // 25 // 400) * 400
    ea_n = NW * ew_a
    src_a, src_b = edge_index[0, :ea_n], edge_index[0, ea_n:]
    dst_a, dst_b = edge_index[1, :ea_n], edge_index[1, ea_n:]
    ea_a, ea_b = edge_attr[:ea_n], edge_attr[ea_n:]

    conv1, conv2 = params["convs"]
    head = params["head"]
    h = head["W1"].shape[0]

    gather_a = _make_gather_add(n, ea_n, h)
    gather_b = _make_gather_add(n, e - ea_n, h)
    scatter_a = _make_scatter_add(n, ea_n, h)
    scatter_b = _make_scatter_add(n, e - ea_n, h)

    def layer(xin, conv):
        ewt = conv["edge"]["W1"]
        dd = xin.shape[1]
        at, bt = _proj2(xin, ewt[:dd], ewt[dd:2 * dd])
        gb = gather_b(at, bt, src_b, dst_b)
        ga = gather_a(at, bt, src_a, dst_a)
        mb = _edge_mlp(gb, ea_b, ewt[2 * dd:], conv["edge"]["b1"],
                       conv["edge"]["W2"], conv["edge"]["b2"])
        ma = _edge_mlp(ga, ea_a, ewt[2 * dd:], conv["edge"]["b1"],
                       conv["edge"]["W2"], conv["edge"]["b2"])
        pb = scatter_b(mb, dst_b)
        pa = scatter_a(ma, dst_a)
        return pa, pb

    p1a, p1b = layer(x, conv1)
    nw1 = conv1["node"]["W1"]
    ew2 = conv2["edge"]["W1"]
    h1, a2t, b2t = _node_mid(
        x, p1a, p1b, nw1[:d], nw1[d:], conv1["node"]["b1"],
        conv1["node"]["W2"], conv1["node"]["b2"],
        ew2[:h], ew2[h:2 * h])

    g2b = gather_b(a2t, b2t, src_b, dst_b)
    g2a = gather_a(a2t, b2t, src_a, dst_a)
    m2b = _edge_mlp(g2b, ea_b, ew2[2 * h:], conv2["edge"]["b1"],
                    conv2["edge"]["W2"], conv2["edge"]["b2"])
    m2a = _edge_mlp(g2a, ea_a, ew2[2 * h:], conv2["edge"]["b1"],
                    conv2["edge"]["W2"], conv2["edge"]["b2"])
    p2b = scatter_b(m2b, dst_b)
    p2a = scatter_a(m2a, dst_a)

    nw2 = conv2["node"]["W1"]
    out = _node_final(
        h1, p2a, p2b, nw2[:h], nw2[h:], conv2["node"]["b1"],
        conv2["node"]["W2"], conv2["node"]["b2"],
        head["W1"], head["b1"], head["W2"], head["b2"])
    return out


# speedup vs baseline: 1.0010x; 1.0005x over previous
import functools

import jax
import jax.numpy as jnp
from jax import lax
from jax.experimental import pallas as pl
from jax.experimental.pallas import tpu as pltpu
from jax.experimental.pallas import tpu_sc as plsc

NC = 2
NS = 16
NW = NC * NS
LANES = 16

_F32 = jnp.float32


def _sc_mesh():
    return plsc.VectorSubcoreMesh(
        core_axis_name="c", subcore_axis_name="s",
        num_cores=NC, num_subcores=NS)


NB = 5


def _pick_chunk(ew, kmax, step=8):
    for k in range(kmax - kmax % step, 0, -step):
        if ew % k == 0 and (ew // k) % NB == 0:
            return k
    raise ValueError(ew)


@functools.lru_cache(maxsize=None)
def _make_gather_add(n, e, h):
    ew = e // NW
    k = _pick_chunk(ew, 504 // NB)
    nchunk = ew // k
    ngroups = nchunk // NB

    @functools.partial(
        pl.kernel,
        mesh=_sc_mesh(),
        out_type=jax.ShapeDtypeStruct((e, h), _F32),
        scratch_types=[
            pltpu.VMEM((NB, k), jnp.int32),
            pltpu.VMEM((NB, k), jnp.int32),
            pltpu.VMEM((NB, k, h), _F32),
            pltpu.VMEM((NB, k, h), _F32),
            pltpu.SemaphoreType.DMA((NB,)),
            pltpu.SemaphoreType.DMA((NB,)),
            pltpu.SemaphoreType.DMA((NB,)),
            pltpu.SemaphoreType.DMA((NB,)),
            pltpu.SemaphoreType.DMA((NB,)),
        ],
    )
    def gather_add(a_hbm, b_hbm, src_hbm, dst_hbm, out_hbm,
                   sidx, didx, arows, brows,
                   sem_s, sem_d, sem_a, sem_b, sem_o):
        wid = lax.axis_index("s") * NC + lax.axis_index("c")
        base = wid * ew

        def group(g, carry):
            cs, cd = [], []
            for b in range(NB):
                off = base + (g * NB + b) * k
                cs.append(pltpu.async_copy(
                    src_hbm.at[pl.ds(off, k)], sidx.at[b], sem_s.at[b]))
                cd.append(pltpu.async_copy(
                    dst_hbm.at[pl.ds(off, k)], didx.at[b], sem_d.at[b]))
            ga, gb = [], []
            for b in range(NB):
                cs[b].wait()
                cd[b].wait()

                @pl.when(g > 0)
                def _(b=b):
                    off_prev = base + ((g - 1) * NB + b) * k
                    pltpu.make_async_copy(
                        arows.at[b], out_hbm.at[pl.ds(off_prev, k)],
                        sem_o.at[b]).wait()

                ga.append(pltpu.async_copy(
                    a_hbm.at[sidx.at[b]], arows.at[b], sem_a.at[b]))
                gb.append(pltpu.async_copy(
                    b_hbm.at[didx.at[b]], brows.at[b], sem_b.at[b]))
            for b in range(NB):
                ga[b].wait()
                gb[b].wait()

                def row(r, c2, b=b):
                    for j in range(h // LANES):
                        sl = pl.ds(j * LANES, LANES)
                        plsc.addupdate(arows.at[b, r, sl], brows[b, r, sl])
                    return c2

                lax.fori_loop(0, k, row, 0, unroll=2)
                off = base + (g * NB + b) * k
                pltpu.async_copy(
                    arows.at[b], out_hbm.at[pl.ds(off, k)], sem_o.at[b])
            return carry

        lax.fori_loop(0, ngroups, group, 0)
        for b in range(NB):
            off_last = base + ((ngroups - 1) * NB + b) * k
            pltpu.make_async_copy(
                arows.at[b], out_hbm.at[pl.ds(off_last, k)],
                sem_o.at[b]).wait()

    return gather_add


@functools.lru_cache(maxsize=None)
def _make_scatter_add(n, e, h):
    ew = e // NW
    ks = _pick_chunk(ew, 72)
    nchunk = ew // ks
    zr = min(ks, 40)
    rpt = -(-(-(-n // NS)) // zr) * zr
    last_rows = n - (NS - 1) * rpt
    assert rpt % zr == 0 and last_rows % zr == 0 and last_rows > 0
    assert rpt % 8 == 0 and last_rows % 8 == 0

    @functools.partial(
        pl.kernel,
        mesh=_sc_mesh(),
        out_type=jax.ShapeDtypeStruct((NC, n, h), _F32),
        scratch_types=[
            pltpu.VMEM((NB, ks), jnp.int32),
            pltpu.VMEM((NB, ks, h), _F32),
            pltpu.VMEM_SHARED((n, h), _F32),
            pltpu.SemaphoreType.DMA((NB,)),
            pltpu.SemaphoreType.DMA((NB,)),
        ],
    )
    def scatter_add(m_hbm, dst_hbm, out_hbm, didx, rows, acc,
                    sem_i, sem_r):
        cid = lax.axis_index("c")
        sid = lax.axis_index("s")
        wid = sid * NC + cid

        zero = jnp.zeros((LANES,), _F32)
        zbuf = rows.at[0, pl.ds(0, zr)]

        def zrow(r, carry):
            for j in range(h // LANES):
                rows[0, r, pl.ds(j * LANES, LANES)] = zero
            return carry

        lax.fori_loop(0, zr, zrow, 0)
        tb = sid * rpt
        nrows = jnp.where(sid == NS - 1, last_rows, rpt)
        ncopies = nrows // zr

        def zcopy(i, carry):
            pltpu.sync_copy(zbuf, acc.at[pl.ds(tb + i * zr, zr)])
            return carry

        lax.fori_loop(0, ncopies, zcopy, 0)
        plsc.subcore_barrier()

        base = wid * ew
        ngroups = nchunk // NB

        def group(g, carry):
            ci_, cr_ = [], []
            for b in range(NB):
                off = base + (g * NB + b) * ks
                ci_.append(pltpu.async_copy(
                    dst_hbm.at[pl.ds(off, ks)], didx.at[b], sem_i.at[b]))
                cr_.append(pltpu.async_copy(
                    m_hbm.at[pl.ds(off, ks)], rows.at[b], sem_r.at[b]))
            for b in range(NB):
                ci_[b].wait()
                cr_[b].wait()
                pltpu.sync_copy(rows.at[b], acc.at[didx.at[b]], add=True)
            return carry

        lax.fori_loop(0, ngroups, group, 0)
        plsc.subcore_barrier()

        def ocopy(i, carry):
            sl = pl.ds(tb + i * zr, zr)
            pltpu.sync_copy(acc.at[sl], out_hbm.at[cid].at[sl])
            return carry

        lax.fori_loop(0, ncopies, ocopy, 0)

    return scatter_add


def _proj2_body(x_ref, ws_ref, wd_ref, a_ref, b_ref):
    x = x_ref[...]
    a_ref[...] = jnp.dot(x, ws_ref[...], preferred_element_type=_F32)
    b_ref[...] = jnp.dot(x, wd_ref[...], preferred_element_type=_F32)


def _proj2(x, ws, wd, bn=2000):
    n, d = x.shape
    h = ws.shape[1]
    grid = n // bn
    return pl.pallas_call(
        _proj2_body,
        grid=(grid,),
        in_specs=[
            pl.BlockSpec((bn, d), lambda i: (i, 0)),
            pl.BlockSpec((d, h), lambda i: (0, 0)),
            pl.BlockSpec((d, h), lambda i: (0, 0)),
        ],
        out_specs=[
            pl.BlockSpec((bn, h), lambda i: (i, 0)),
            pl.BlockSpec((bn, h), lambda i: (i, 0)),
        ],
        out_shape=[
            jax.ShapeDtypeStruct((n, h), _F32),
            jax.ShapeDtypeStruct((n, h), _F32),
        ],
    )(x, ws, wd)


def _edge_mlp_body(g_ref, ea_ref, w1e_ref, b1_ref, w2_ref, b2_ref, out_ref):
    c = jnp.dot(ea_ref[...], w1e_ref[...], preferred_element_type=_F32)
    m1 = jnp.maximum(g_ref[...] + c + b1_ref[...], 0.0)
    m2 = jnp.dot(m1, w2_ref[...], preferred_element_type=_F32)
    out_ref[...] = jnp.maximum(m2 + b2_ref[...], 0.0)


def _edge_mlp(g, ea, w1e, b1, w2, b2, be=3200):
    e, h = g.shape
    de = ea.shape[1]
    grid = e // be
    return pl.pallas_call(
        _edge_mlp_body,
        grid=(grid,),
        in_specs=[
            pl.BlockSpec((be, h), lambda i: (i, 0)),
            pl.BlockSpec((be, de), lambda i: (i, 0)),
            pl.BlockSpec((de, h), lambda i: (0, 0)),
            pl.BlockSpec((1, h), lambda i: (0, 0)),
            pl.BlockSpec((h, h), lambda i: (0, 0)),
            pl.BlockSpec((1, h), lambda i: (0, 0)),
        ],
        out_specs=pl.BlockSpec((be, h), lambda i: (i, 0)),
        out_shape=jax.ShapeDtypeStruct((e, h), _F32),
    )(g, ea, w1e, b1.reshape(1, h), w2, b2.reshape(1, h))


def _node_mid_body(x_ref, p_ref, q_ref, w1a_ref, w1b_ref, nb1_ref, w2_ref,
                   nb2_ref, ws2_ref, wd2_ref, h_ref, a2_ref, b2_ref):
    agg = (p_ref[0] + p_ref[1]) + (q_ref[0] + q_ref[1])
    t = jnp.dot(x_ref[...], w1a_ref[...], preferred_element_type=_F32)
    t = t + jnp.dot(agg, w1b_ref[...], preferred_element_type=_F32)
    t = jnp.maximum(t + nb1_ref[...], 0.0)
    hh = jnp.dot(t, w2_ref[...], preferred_element_type=_F32) + nb2_ref[...]
    h_ref[...] = hh
    a2_ref[...] = jnp.dot(hh, ws2_ref[...], preferred_element_type=_F32)
    b2_ref[...] = jnp.dot(hh, wd2_ref[...], preferred_element_type=_F32)


def _node_mid(x, p, q, w1a, w1b, nb1, w2, nb2, ws2, wd2, bn=2000):
    n, d = x.shape
    h = w1b.shape[0]
    grid = n // bn
    return pl.pallas_call(
        _node_mid_body,
        grid=(grid,),
        in_specs=[
            pl.BlockSpec((bn, d), lambda i: (i, 0)),
            pl.BlockSpec((NC, bn, h), lambda i: (0, i, 0)),
            pl.BlockSpec((NC, bn, h), lambda i: (0, i, 0)),
            pl.BlockSpec((d, h), lambda i: (0, 0)),
            pl.BlockSpec((h, h), lambda i: (0, 0)),
            pl.BlockSpec((1, h), lambda i: (0, 0)),
            pl.BlockSpec((h, h), lambda i: (0, 0)),
            pl.BlockSpec((1, h), lambda i: (0, 0)),
            pl.BlockSpec((h, h), lambda i: (0, 0)),
            pl.BlockSpec((h, h), lambda i: (0, 0)),
        ],
        out_specs=[
            pl.BlockSpec((bn, h), lambda i: (i, 0)),
            pl.BlockSpec((bn, h), lambda i: (i, 0)),
            pl.BlockSpec((bn, h), lambda i: (i, 0)),
        ],
        out_shape=[
            jax.ShapeDtypeStruct((n, h), _F32),
            jax.ShapeDtypeStruct((n, h), _F32),
            jax.ShapeDtypeStruct((n, h), _F32),
        ],
    )(x, p, q, w1a, w1b, nb1.reshape(1, h), w2, nb2.reshape(1, h), ws2, wd2)


def _node_final_body(x_ref, p_ref, q_ref, w1a_ref, w1b_ref, nb1_ref, w2_ref,
                     nb2_ref, hw1_ref, hb1_ref, hw2_ref, hb2_ref, out_ref):
    agg = (p_ref[0] + p_ref[1]) + (q_ref[0] + q_ref[1])
    t = jnp.dot(x_ref[...], w1a_ref[...], preferred_element_type=_F32)
    t = t + jnp.dot(agg, w1b_ref[...], preferred_element_type=_F32)
    t = jnp.maximum(t + nb1_ref[...], 0.0)
    hh = jnp.dot(t, w2_ref[...], preferred_element_type=_F32) + nb2_ref[...]
    z = jnp.maximum(
        jnp.dot(hh, hw1_ref[...], preferred_element_type=_F32) + hb1_ref[...],
        0.0)
    out_ref[...] = (
        jnp.dot(z, hw2_ref[...], preferred_element_type=_F32) + hb2_ref[...])


def _node_final(x, p, q, w1a, w1b, nb1, w2, nb2, hw1, hb1, hw2, hb2, bn=2000):
    n, d = x.shape
    h = w1b.shape[0]
    grid = n // bn
    return pl.pallas_call(
        _node_final_body,
        grid=(grid,),
        in_specs=[
            pl.BlockSpec((bn, d), lambda i: (i, 0)),
            pl.BlockSpec((NC, bn, h), lambda i: (0, i, 0)),
            pl.BlockSpec((NC, bn, h), lambda i: (0, i, 0)),
            pl.BlockSpec((d, h), lambda i: (0, 0)),
            pl.BlockSpec((h, h), lambda i: (0, 0)),
            pl.BlockSpec((1, h), lambda i: (0, 0)),
            pl.BlockSpec((h, h), lambda i: (0, 0)),
            pl.BlockSpec((1, h), lambda i: (0, 0)),
            pl.BlockSpec((h, h), lambda i: (0, 0)),
            pl.BlockSpec((1, h), lambda i: (0, 0)),
            pl.BlockSpec((h, 1), lambda i: (0, 0)),
            pl.BlockSpec((1, 1), lambda i: (0, 0)),
        ],
        out_specs=pl.BlockSpec((bn, 1), lambda i: (i, 0)),
        out_shape=jax.ShapeDtypeStruct((n, 1), _F32),
    )(x, p, q, w1a, w1b, nb1.reshape(1, h), w2, nb2.reshape(1, h),
      hw1, hb1.reshape(1, h), hw2, hb2.reshape(1, 1))


def kernel(x, edge_index, edge_attr, params):
    n, d = x.shape
    e = edge_index.shape[1]
    ew_tot = e // NW
    ew_a = ((ew_tot * 16) // 25 // 400) * 400
    ea_n = NW * ew_a
    src_a, src_b = edge_index[0, :ea_n], edge_index[0, ea_n:]
    dst_a, dst_b = edge_index[1, :ea_n], edge_index[1, ea_n:]
    ea_a, ea_b = edge_attr[:ea_n], edge_attr[ea_n:]

    conv1, conv2 = params["convs"]
    head = params["head"]
    h = head["W1"].shape[0]

    gather_a = _make_gather_add(n, ea_n, h)
    gather_b = _make_gather_add(n, e - ea_n, h)
    scatter_a = _make_scatter_add(n, ea_n, h)
    scatter_b = _make_scatter_add(n, e - ea_n, h)

    def layer(xin, conv):
        ewt = conv["edge"]["W1"]
        dd = xin.shape[1]
        at, bt = _proj2(xin, ewt[:dd], ewt[dd:2 * dd])
        gb = gather_b(at, bt, src_b, dst_b)
        ga = gather_a(at, bt, src_a, dst_a)
        mb = _edge_mlp(gb, ea_b, ewt[2 * dd:], conv["edge"]["b1"],
                       conv["edge"]["W2"], conv["edge"]["b2"])
        ma = _edge_mlp(ga, ea_a, ewt[2 * dd:], conv["edge"]["b1"],
                       conv["edge"]["W2"], conv["edge"]["b2"])
        pb = scatter_b(mb, dst_b)
        pa = scatter_a(ma, dst_a)
        return pa, pb

    p1a, p1b = layer(x, conv1)
    nw1 = conv1["node"]["W1"]
    ew2 = conv2["edge"]["W1"]
    h1, a2t, b2t = _node_mid(
        x, p1a, p1b, nw1[:d], nw1[d:], conv1["node"]["b1"],
        conv1["node"]["W2"], conv1["node"]["b2"],
        ew2[:h], ew2[h:2 * h])

    g2b = gather_b(a2t, b2t, src_b, dst_b)
    g2a = gather_a(a2t, b2t, src_a, dst_a)
    m2b = _edge_mlp(g2b, ea_b, ew2[2 * h:], conv2["edge"]["b1"],
                    conv2["edge"]["W2"], conv2["edge"]["b2"])
    m2a = _edge_mlp(g2a, ea_a, ew2[2 * h:], conv2["edge"]["b1"],
                    conv2["edge"]["W2"], conv2["edge"]["b2"])
    p2b = scatter_b(m2b, dst_b)
    p2a = scatter_a(m2a, dst_a)

    nw2 = conv2["node"]["W1"]
    out = _node_final(
        h1, p2a, p2b, nw2[:h], nw2[h:], conv2["node"]["b1"],
        conv2["node"]["W2"], conv2["node"]["b2"],
        head["W1"], head["b1"], head["W2"], head["b2"])
    return out
